# baseline (speedup 1.0000x reference)
import functools

import numpy as np
import jax
import jax.numpy as jnp
from jax import lax
from jax.experimental import pallas as pl
from jax.experimental.pallas import tpu as pltpu
from jax.experimental.pallas import tpu_sc as plsc

N = 10000
E = 320000
HID = 128
NC = 2
NS = 16
NW = NC * NS
F = HID // NS
FMAX = 3.4028235e38
NEG_INF = float("-inf")

_mesh = plsc.VectorSubcoreMesh(
    core_axis_name="c", subcore_axis_name="s", num_cores=NC, num_subcores=NS)


def _mm(a, b):
    return jnp.dot(a, b, preferred_element_type=jnp.float32)


@functools.cache
def _build_gather(n, e, gc):
    ec = e // NW
    nbuf = 4
    sc_sz = nbuf * gc
    nsuper = ec // sc_sz
    rem = ec - nsuper * sc_sz

    @functools.partial(
        pl.kernel,
        out_type=jax.ShapeDtypeStruct((e, HID), jnp.float32),
        mesh=_mesh,
        scratch_types=[
            pltpu.VMEM((ec,), jnp.int32),
            pltpu.VMEM((ec,), jnp.int32),
            pltpu.VMEM((nbuf, gc, HID), jnp.float32),
            pltpu.SemaphoreType.DMA,
            pltpu.SemaphoreType.DMA,
            pltpu.SemaphoreType.DMA,
            pltpu.SemaphoreType.DMA,
        ],
        compiler_params=pltpu.CompilerParams(needs_layout_passes=False),
        name="sc_gather_add",
    )
    def gather(a_hbm, b_hbm, dst_hbm, src_hbm, z_hbm,
               didx, sidx, zbufs, sem_i, sem_a, sem_b, sem_w):
        w = lax.axis_index("c") * NS + lax.axis_index("s")
        base0 = w * ec
        di = pltpu.async_copy(dst_hbm.at[pl.ds(base0, ec)], didx, sem_i)
        si = pltpu.async_copy(src_hbm.at[pl.ds(base0, ec)], sidx, sem_i)
        di.wait()
        si.wait()

        def superchunk(i, carry):
            off = i * sc_sz

            @pl.when(i > 0)
            def _():
                for b in range(nbuf):
                    pltpu.make_async_copy(
                        zbufs.at[b],
                        z_hbm.at[pl.ds(base0 + off - sc_sz + b * gc, gc)],
                        sem_w).wait()

            cps = [pltpu.async_copy(
                a_hbm.at[didx.at[pl.ds(off + b * gc, gc)]],
                zbufs.at[b], sem_a) for b in range(nbuf)]
            for cp in cps:
                cp.wait()
            cps = [pltpu.async_copy(
                b_hbm.at[sidx.at[pl.ds(off + b * gc, gc)]],
                zbufs.at[b], sem_b, add=True) for b in range(nbuf)]
            for cp in cps:
                cp.wait()
            for b in range(nbuf):
                pltpu.async_copy(
                    zbufs.at[b],
                    z_hbm.at[pl.ds(base0 + off + b * gc, gc)], sem_w)
            return carry

        lax.fori_loop(0, nsuper, superchunk, 0)
        for b in range(nbuf):
            pltpu.make_async_copy(
                zbufs.at[b],
                z_hbm.at[pl.ds(base0 + (nsuper - 1) * sc_sz + b * gc, gc)],
                sem_w).wait()
        off = nsuper * sc_sz
        while off < ec:
            sz = min(gc, ec - off)
            zb = zbufs.at[0].at[pl.ds(0, sz)] if sz != gc else zbufs.at[0]
            pltpu.async_copy(a_hbm.at[didx.at[pl.ds(off, sz)]], zb,
                             sem_a).wait()
            pltpu.async_copy(b_hbm.at[sidx.at[pl.ds(off, sz)]], zb,
                             sem_b, add=True).wait()
            pltpu.async_copy(zb, z_hbm.at[pl.ds(base0 + off, sz)],
                             sem_w).wait()
            off += sz

    return gather


@functools.cache
def _build_scatter(n, e, schunk):
    eh = e // 2
    nchunk = eh // schunk
    ngroups = schunk // 16

    @functools.partial(
        pl.kernel,
        out_type=jax.ShapeDtypeStruct((2, HID, n), jnp.float32),
        mesh=_mesh,
        scratch_types=[
            pltpu.VMEM((schunk,), jnp.int32),
            pltpu.VMEM((schunk,), jnp.int32),
            pltpu.VMEM((F, schunk), jnp.float32),
            pltpu.VMEM((F, schunk), jnp.float32),
            pltpu.VMEM((F, n), jnp.float32),
            pltpu.VMEM((n,), jnp.int32),
            pltpu.VMEM((16,), jnp.int32),
            pltpu.SemaphoreType.DMA,
            pltpu.SemaphoreType.DMA,
        ],
        compiler_params=pltpu.CompilerParams(needs_layout_passes=False),
        name="sc_scatter_max",
    )
    def scatter(m_hbm, dst_hbm, p_hbm, dbuf0, dbuf1, vbuf0, vbuf1, acc,
                cntbuf, flagbuf, sem0, sem1):
        c = lax.axis_index("c")
        s = lax.axis_index("s")
        iota = lax.iota(jnp.int32, 16)
        neg = jnp.full((16,), NEG_INF, jnp.float32)
        ones = jnp.full((16,), 1, jnp.int32)
        zeros = jnp.full((16,), 0, jnp.int32)

        def initb(i, carry):
            for f in range(F):
                acc[f, pl.ds(i * 16, 16)] = neg
            return carry

        lax.fori_loop(0, n // 16, initb, 0)

        base0 = c * eh
        col = s * F
        fsplat = [jnp.full((16,), f, jnp.int32) for f in range(F)]
        bufs = [(dbuf0, vbuf0, sem0), (dbuf1, vbuf1, sem1)]

        def start_fetch(ci, b):
            db, vb, sem = bufs[b]
            base = base0 + ci * schunk
            pltpu.async_copy(dst_hbm.at[pl.ds(base, schunk)], db, sem)
            pltpu.async_copy(m_hbm.at[pl.ds(col, F), pl.ds(base, schunk)],
                             vb, sem)

        def wait_fetch(b):
            db, vb, sem = bufs[b]
            pltpu.make_async_copy(dst_hbm.at[pl.ds(0, schunk)], db,
                                  sem).wait()
            pltpu.make_async_copy(m_hbm.at[pl.ds(0, F), pl.ds(0, schunk)],
                                  vb, sem).wait()

        def process(b):
            dbuf, vbuf, _ = bufs[b]

            def group(g, gcarry):
                d16 = dbuf[pl.ds(g * 16, 16)]
                plsc.store_scatter(cntbuf, [d16], iota)
                back = plsc.load_gather(cntbuf, [d16])
                losers = back != iota
                plsc.store_scatter(flagbuf, [zeros], zeros)
                plsc.store_scatter(flagbuf, [zeros], ones, mask=losers)
                for f in range(F):
                    vals = vbuf[f, pl.ds(g * 16, 16)]
                    cur = plsc.load_gather(acc, [fsplat[f], d16])
                    plsc.store_scatter(acc, [fsplat[f], d16],
                                       jnp.maximum(cur, vals))

                @pl.when(flagbuf[pl.ds(0, 16)][0] > 0)
                def _():
                    vals = [vbuf[f, pl.ds(g * 16, 16)] for f in range(F)]
                    plsc.store_scatter(cntbuf, [d16],
                                       jnp.full((16,), -1, jnp.int32),
                                       mask=losers)
                    mk = plsc.load_gather(cntbuf, [d16])
                    pend0 = mk == -1

                    def cond(carry2):
                        pend, r = carry2
                        return (jnp.max(jnp.where(pend, ones, zeros)) > 0
                                ) & (r < 32)

                    def body(carry2):
                        pend, r = carry2
                        pnew = pend & (iota < 0)
                        for f in range(F):
                            cur = plsc.load_gather(acc, [fsplat[f], d16],
                                                   mask=pend)
                            nv = jnp.maximum(cur, vals[f])
                            plsc.store_scatter(acc, [fsplat[f], d16], nv,
                                               mask=pend)
                            chk = plsc.load_gather(acc, [fsplat[f], d16],
                                                   mask=pend)
                            pnew = pnew | (pend & (chk < nv))
                        return pnew, r + 1

                    lax.while_loop(cond, body, (pend0, 0))
                return gcarry

            lax.fori_loop(0, ngroups, group, 0)

        start_fetch(0, 0)

        def pair(j, carry):
            c0 = 2 * j
            wait_fetch(0)
            start_fetch(jnp.minimum(c0 + 1, nchunk - 1), 1)
            process(0)
            wait_fetch(1)
            start_fetch(jnp.minimum(c0 + 2, nchunk - 1), 0)
            process(1)
            return carry

        lax.fori_loop(0, nchunk // 2, pair, 0)
        wait_fetch(0)
        pltpu.sync_copy(acc, p_hbm.at[c, pl.ds(col, F), :])

    return scatter


@functools.cache
def _build_prep0(n, nb):
    bs = n // nb

    def body(x_ref, w1, b1, w2, b2, wa, ba, wb, a_out, b_out):
        x = x_ref[...]
        h = _mm(jnp.maximum(_mm(x, w1[...]) + b1[...], 0.0), w2[...]) + b2[...]
        a_out[...] = _mm(h, wa[...]) + ba[...]
        b_out[...] = _mm(h, wb[...])

    full = pl.BlockSpec((HID, HID), lambda i: (0, 0))
    vec = pl.BlockSpec((1, HID), lambda i: (0, 0))
    return pl.pallas_call(
        body,
        grid=(nb,),
        in_specs=[pl.BlockSpec((bs, HID), lambda i: (i, 0)),
                  full, vec, full, vec, full, vec, full],
        out_specs=[pl.BlockSpec((bs, HID), lambda i: (i, 0)),
                   pl.BlockSpec((bs, HID), lambda i: (i, 0))],
        out_shape=[jax.ShapeDtypeStruct((n, HID), jnp.float32),
                   jax.ShapeDtypeStruct((n, HID), jnp.float32)],
    )


def _tdot(a, b):
    return lax.dot_general(a, b, (((0,), (0,)), ((), ())),
                           preferred_element_type=jnp.float32)


@functools.cache
def _build_edge_mlp(e, bs):
    def body(z_ref, w2, m_ref):
        m_ref[...] = lax.dot_general(
            w2[...], jnp.maximum(z_ref[...], 0.0),
            (((0,), (1,)), ((), ())), preferred_element_type=jnp.float32)

    return pl.pallas_call(
        body,
        grid=(e // bs,),
        in_specs=[pl.BlockSpec((bs, HID), lambda i: (i, 0)),
                  pl.BlockSpec((HID, HID), lambda i: (0, 0))],
        out_specs=pl.BlockSpec((HID, bs), lambda i: (0, i)),
        out_shape=jax.ShapeDtypeStruct((HID, e), jnp.float32),
    )


def _agg_to_h_t(p_blk, b2col):
    aggb = jnp.max(p_blk, axis=0) + b2col
    finite = (aggb >= -FMAX) & (aggb <= FMAX)
    return jnp.maximum(jnp.where(finite, aggb, 0.0), 0.0)


@functools.cache
def _build_consume(n, nb):
    del nb

    def body(p_ref, b2p, wa, ba, wb, a_out, b_out):
        ht = _agg_to_h_t(p_ref[...], b2p[...])
        a_out[...] = _tdot(ht, wa[...]) + ba[...]
        b_out[...] = _tdot(ht, wb[...])

    return pl.pallas_call(
        body,
        out_shape=[jax.ShapeDtypeStruct((n, HID), jnp.float32),
                   jax.ShapeDtypeStruct((n, HID), jnp.float32)],
    )


@functools.cache
def _build_consume_final(n, nb):
    del nb

    def body(p_ref, b2p, sum_out):
        ht = _agg_to_h_t(p_ref[...], b2p[...])
        sum_out[...] = jnp.sum(ht, axis=1, keepdims=True)

    return pl.pallas_call(
        body,
        out_shape=jax.ShapeDtypeStruct((HID, 1), jnp.float32),
    )


def _heads_body(fsum, ssum, wfp, bfp, wsp, bsp, fw1a, fw1b, fb1, fw2, fb2,
                tmpl, pw1, pb1, pw2, pb2, tw1, tb1, tw2, tb2,
                ew1a, ew1b, eb1, ew2, eb2, exw, exb, etw, etb, ohi, ohj,
                pos_out, typ_out, ex_out, et_out):
    inv_n = 1.0 / N
    fg = _tdot(fsum[...] * inv_n, wfp[...]) + bfp[...]
    sg = _tdot(ssum[...] * inv_n, wsp[...]) + bsp[...]
    pre = jnp.maximum(_mm(fg, fw1a[...]) + _mm(sg, fw1b[...]) + fb1[...], 0.0)
    fused = _mm(pre, fw2[...]) + fb2[...]
    emb = tmpl[...] + fused
    ph = jnp.maximum(_mm(emb, pw1[...]) + pb1[...], 0.0)
    pos_out[...] = _mm(ph, pw2[...]) + pb2[...]
    th = jnp.maximum(_mm(emb, tw1[...]) + tb1[...], 0.0)
    typ_out[...] = _mm(th, tw2[...]) + tb2[...]
    pi = _mm(ohi[...], emb)
    pj = _mm(ohj[...], emb)
    e1 = jnp.maximum(_mm(pi, ew1a[...]) + _mm(pj, ew1b[...]) + eb1[...], 0.0)
    enc = jnp.maximum(_mm(e1, ew2[...]) + eb2[...], 0.0)
    ex_out[...] = _mm(enc, exw[...]) + exb[...]
    et_out[...] = _mm(enc, etw[...]) + etb[...]


@functools.cache
def _build_heads():
    return pl.pallas_call(
        _heads_body,
        out_shape=[jax.ShapeDtypeStruct((6, 2), jnp.float32),
                   jax.ShapeDtypeStruct((6, 2), jnp.float32),
                   jax.ShapeDtypeStruct((15, 1), jnp.float32),
                   jax.ShapeDtypeStruct((15, 16), jnp.float32)],
    )


def _row(v):
    return v.reshape(1, -1)


def _encoder(x, edge_index, enc, n=N, e=E, gc=128, schunk=640, nb=10):
    src = edge_index[0]
    dst = edge_index[1]
    prep = []
    for (cw1, cb1, cw2, cb2) in enc['convs']:
        prep.append((cw1[:HID] - cw1[HID:], _row(cb1), cw1[HID:], cw2,
                     cb2.reshape(-1, 1)))
    w1, b1, w2, b2 = enc['ne']
    a, b = _build_prep0(n, nb)(x, w1, _row(b1), w2, _row(b2),
                               prep[0][0], prep[0][1], prep[0][2])
    gather = _build_gather(n, e, gc)
    edge_mlp = _build_edge_mlp(e, 512)
    scatter = _build_scatter(n, e, schunk)
    for i in range(3):
        z = gather(a, b, dst, src)
        m = edge_mlp(z, prep[i][3])
        p = scatter(m, dst)
        if i < 2:
            a, b = _build_consume(n, nb)(p, prep[i][4], prep[i + 1][0],
                                         prep[i + 1][1], prep[i + 1][2])
        else:
            return _build_consume_final(n, nb)(p, prep[i][4])


def kernel(front_x, front_edge_index, front_edge_attr,
           side_x, side_edge_index, side_edge_attr, params):
    fsum = _encoder(front_x, front_edge_index, params['front'])
    ssum = _encoder(side_x, side_edge_index, params['side'])
    fus_w1, fus_b1, fus_w2, fus_b2 = params['fusion']
    pw1, pb1, pw2, pb2 = params['pos']
    tw1, tb1, tw2, tb2 = params['type']
    ew1, eb1, ew2, eb2 = params['edge']['enc']
    ii, jj = np.triu_indices(6, k=1)
    ohi = jnp.asarray(np.eye(6, dtype=np.float32)[ii])
    ohj = jnp.asarray(np.eye(6, dtype=np.float32)[jj])
    pos, typ, exist, etype = _build_heads()(
        fsum, ssum,
        params['front_pool'][0], _row(params['front_pool'][1]),
        params['side_pool'][0], _row(params['side_pool'][1]),
        fus_w1[:HID], fus_w1[HID:], _row(fus_b1), fus_w2, _row(fus_b2),
        params['templates'][:6],
        pw1, _row(pb1), pw2, _row(pb2),
        tw1, _row(tb1), tw2, _row(tb2),
        ew1[:HID], ew1[HID:], _row(eb1), ew2, _row(eb2),
        params['edge']['exist'][0], _row(params['edge']['exist'][1]),
        params['edge']['type'][0], _row(params['edge']['type'][1]),
        ohi, ohj)
    return pos, typ, exist, etype


# speedup vs baseline: 1.5192x; 1.5192x over previous
import functools

import numpy as np
import jax
import jax.numpy as jnp
from jax import lax
from jax.experimental import pallas as pl
from jax.experimental.pallas import tpu as pltpu
from jax.experimental.pallas import tpu_sc as plsc

N = 10000
E = 320000
HID = 128
NC = 2
NS = 16
NW = NC * NS
F = HID // NS
FMAX = 3.4028235e38
NEG_INF = float("-inf")

_mesh = plsc.VectorSubcoreMesh(
    core_axis_name="c", subcore_axis_name="s", num_cores=NC, num_subcores=NS)


def _mm(a, b):
    return jnp.dot(a, b, preferred_element_type=jnp.float32)


@functools.cache
def _build_gather(n, e, gc):
    ec = e // NW
    nbuf = 4
    sc_sz = nbuf * gc
    nsuper = ec // sc_sz
    rem = ec - nsuper * sc_sz

    @functools.partial(
        pl.kernel,
        out_type=jax.ShapeDtypeStruct((e, HID), jnp.float32),
        mesh=_mesh,
        scratch_types=[
            pltpu.VMEM((ec,), jnp.int32),
            pltpu.VMEM((ec,), jnp.int32),
            pltpu.VMEM((nbuf, gc, HID), jnp.float32),
            pltpu.SemaphoreType.DMA,
            pltpu.SemaphoreType.DMA,
            pltpu.SemaphoreType.DMA,
            pltpu.SemaphoreType.DMA,
        ],
        compiler_params=pltpu.CompilerParams(needs_layout_passes=False),
        name="sc_gather_add",
    )
    def gather(a_hbm, b_hbm, dst_hbm, src_hbm, z_hbm,
               didx, sidx, zbufs, sem_i, sem_a, sem_b, sem_w):
        w = lax.axis_index("c") * NS + lax.axis_index("s")
        base0 = w * ec
        di = pltpu.async_copy(dst_hbm.at[pl.ds(base0, ec)], didx, sem_i)
        si = pltpu.async_copy(src_hbm.at[pl.ds(base0, ec)], sidx, sem_i)
        di.wait()
        si.wait()

        def superchunk(i, carry):
            off = i * sc_sz

            @pl.when(i > 0)
            def _():
                for b in range(nbuf):
                    pltpu.make_async_copy(
                        zbufs.at[b],
                        z_hbm.at[pl.ds(base0 + off - sc_sz + b * gc, gc)],
                        sem_w).wait()

            cps = [pltpu.async_copy(
                a_hbm.at[didx.at[pl.ds(off + b * gc, gc)]],
                zbufs.at[b], sem_a) for b in range(nbuf)]
            for cp in cps:
                cp.wait()
            cps = [pltpu.async_copy(
                b_hbm.at[sidx.at[pl.ds(off + b * gc, gc)]],
                zbufs.at[b], sem_b, add=True) for b in range(nbuf)]
            for cp in cps:
                cp.wait()
            for b in range(nbuf):
                pltpu.async_copy(
                    zbufs.at[b],
                    z_hbm.at[pl.ds(base0 + off + b * gc, gc)], sem_w)
            return carry

        lax.fori_loop(0, nsuper, superchunk, 0)
        for b in range(nbuf):
            pltpu.make_async_copy(
                zbufs.at[b],
                z_hbm.at[pl.ds(base0 + (nsuper - 1) * sc_sz + b * gc, gc)],
                sem_w).wait()
        off = nsuper * sc_sz
        while off < ec:
            sz = min(gc, ec - off)
            zb = zbufs.at[0].at[pl.ds(0, sz)] if sz != gc else zbufs.at[0]
            pltpu.async_copy(a_hbm.at[didx.at[pl.ds(off, sz)]], zb,
                             sem_a).wait()
            pltpu.async_copy(b_hbm.at[sidx.at[pl.ds(off, sz)]], zb,
                             sem_b, add=True).wait()
            pltpu.async_copy(zb, z_hbm.at[pl.ds(base0 + off, sz)],
                             sem_w).wait()
            off += sz

    return gather


@functools.cache
def _build_scatter(n, e, schunk):
    eh = e // 2
    nchunk = eh // schunk
    ngroups = schunk // 16

    @functools.partial(
        pl.kernel,
        out_type=jax.ShapeDtypeStruct((2, HID, n), jnp.float32),
        mesh=_mesh,
        scratch_types=[
            pltpu.VMEM((schunk,), jnp.int32),
            pltpu.VMEM((schunk,), jnp.int32),
            pltpu.VMEM((F, schunk), jnp.float32),
            pltpu.VMEM((F, schunk), jnp.float32),
            pltpu.VMEM((F, n), jnp.float32),
            pltpu.VMEM((n,), jnp.int32),
            pltpu.SemaphoreType.DMA,
            pltpu.SemaphoreType.DMA,
        ],
        compiler_params=pltpu.CompilerParams(needs_layout_passes=False),
        name="sc_scatter_max",
    )
    def scatter(m_hbm, dst_hbm, p_hbm, dbuf0, dbuf1, vbuf0, vbuf1, acc,
                cntbuf, sem0, sem1):
        c = lax.axis_index("c")
        s = lax.axis_index("s")
        iota = lax.iota(jnp.int32, 16)
        neg = jnp.full((16,), NEG_INF, jnp.float32)
        ones = jnp.full((16,), 1, jnp.int32)
        zeros = jnp.full((16,), 0, jnp.int32)

        def initb(i, carry):
            for f in range(F):
                acc[f, pl.ds(i * 16, 16)] = neg
            return carry

        lax.fori_loop(0, n // 16, initb, 0)

        base0 = c * eh
        col = s * F
        fsplat = [jnp.full((16,), f, jnp.int32) for f in range(F)]
        bufs = [(dbuf0, vbuf0, sem0), (dbuf1, vbuf1, sem1)]

        def start_fetch(ci, b):
            db, vb, sem = bufs[b]
            base = base0 + ci * schunk
            pltpu.async_copy(dst_hbm.at[pl.ds(base, schunk)], db, sem)
            pltpu.async_copy(m_hbm.at[pl.ds(col, F), pl.ds(base, schunk)],
                             vb, sem)

        def wait_fetch(b):
            db, vb, sem = bufs[b]
            pltpu.make_async_copy(dst_hbm.at[pl.ds(0, schunk)], db,
                                  sem).wait()
            pltpu.make_async_copy(m_hbm.at[pl.ds(0, F), pl.ds(0, schunk)],
                                  vb, sem).wait()

        def process(b):
            dbuf, vbuf, _ = bufs[b]

            def pairgroup(gp, gcarry):
                datas = []
                lor = None
                for k in range(2):
                    d16 = dbuf[pl.ds(gp * 32 + k * 16, 16)]
                    plsc.store_scatter(cntbuf, [d16], iota)
                    back = plsc.load_gather(cntbuf, [d16])
                    losers = back != iota
                    for f in range(F):
                        vals = vbuf[f, pl.ds(gp * 32 + k * 16, 16)]
                        cur = plsc.load_gather(acc, [fsplat[f], d16])
                        plsc.store_scatter(acc, [fsplat[f], d16],
                                           jnp.maximum(cur, vals))
                    datas.append((d16, losers))
                    lor = losers if k == 0 else lor | losers
                ncol = jnp.max(jnp.where(lor, ones, zeros))

                @pl.when(ncol > 0)
                def _():
                    for k in range(2):
                        d16, losers = datas[k]
                        vals = [vbuf[f, pl.ds(gp * 32 + k * 16, 16)]
                                for f in range(F)]
                        plsc.store_scatter(cntbuf, [d16],
                                           jnp.full((16,), -1, jnp.int32),
                                           mask=losers)
                        mk = plsc.load_gather(cntbuf, [d16])
                        pend0 = mk == -1

                        def cond(carry2):
                            pend, r = carry2
                            return (jnp.max(
                                jnp.where(pend, ones, zeros)) > 0) & (r < 32)

                        def body(carry2):
                            pend, r = carry2
                            pnew = pend & (iota < 0)
                            for f in range(F):
                                cur = plsc.load_gather(
                                    acc, [fsplat[f], d16], mask=pend)
                                nv = jnp.maximum(cur, vals[f])
                                plsc.store_scatter(acc, [fsplat[f], d16],
                                                   nv, mask=pend)
                                chk = plsc.load_gather(
                                    acc, [fsplat[f], d16], mask=pend)
                                pnew = pnew | (pend & (chk < nv))
                            return pnew, r + 1

                        lax.while_loop(cond, body, (pend0, 0))
                return gcarry

            lax.fori_loop(0, ngroups // 2, pairgroup, 0)

        start_fetch(0, 0)

        def pair(j, carry):
            c0 = 2 * j
            wait_fetch(0)
            start_fetch(jnp.minimum(c0 + 1, nchunk - 1), 1)
            process(0)
            wait_fetch(1)
            start_fetch(jnp.minimum(c0 + 2, nchunk - 1), 0)
            process(1)
            return carry

        lax.fori_loop(0, nchunk // 2, pair, 0)
        wait_fetch(0)
        pltpu.sync_copy(acc, p_hbm.at[c, pl.ds(col, F), :])

    return scatter


@functools.cache
def _build_prep0(n, nb):
    bs = n // nb

    def body(x_ref, w1, b1, w2, b2, wa, ba, wb, a_out, b_out):
        x = x_ref[...]
        h = _mm(jnp.maximum(_mm(x, w1[...]) + b1[...], 0.0), w2[...]) + b2[...]
        a_out[...] = _mm(h, wa[...]) + ba[...]
        b_out[...] = _mm(h, wb[...])

    full = pl.BlockSpec((HID, HID), lambda i: (0, 0))
    vec = pl.BlockSpec((1, HID), lambda i: (0, 0))
    return pl.pallas_call(
        body,
        grid=(nb,),
        in_specs=[pl.BlockSpec((bs, HID), lambda i: (i, 0)),
                  full, vec, full, vec, full, vec, full],
        out_specs=[pl.BlockSpec((bs, HID), lambda i: (i, 0)),
                   pl.BlockSpec((bs, HID), lambda i: (i, 0))],
        out_shape=[jax.ShapeDtypeStruct((n, HID), jnp.float32),
                   jax.ShapeDtypeStruct((n, HID), jnp.float32)],
    )


def _tdot(a, b):
    return lax.dot_general(a, b, (((0,), (0,)), ((), ())),
                           preferred_element_type=jnp.float32)


@functools.cache
def _build_edge_mlp(e, bs):
    def body(z_ref, w2, m_ref):
        m_ref[...] = lax.dot_general(
            w2[...], jnp.maximum(z_ref[...], 0.0),
            (((0,), (1,)), ((), ())), preferred_element_type=jnp.float32)

    return pl.pallas_call(
        body,
        grid=(e // bs,),
        in_specs=[pl.BlockSpec((bs, HID), lambda i: (i, 0)),
                  pl.BlockSpec((HID, HID), lambda i: (0, 0))],
        out_specs=pl.BlockSpec((HID, bs), lambda i: (0, i)),
        out_shape=jax.ShapeDtypeStruct((HID, e), jnp.float32),
    )


def _agg_to_h_t(p_blk, b2col):
    aggb = jnp.max(p_blk, axis=0) + b2col
    finite = (aggb >= -FMAX) & (aggb <= FMAX)
    return jnp.maximum(jnp.where(finite, aggb, 0.0), 0.0)


@functools.cache
def _build_consume(n, nb):
    del nb

    def body(p_ref, b2p, wa, ba, wb, a_out, b_out):
        ht = _agg_to_h_t(p_ref[...], b2p[...])
        a_out[...] = _tdot(ht, wa[...]) + ba[...]
        b_out[...] = _tdot(ht, wb[...])

    return pl.pallas_call(
        body,
        out_shape=[jax.ShapeDtypeStruct((n, HID), jnp.float32),
                   jax.ShapeDtypeStruct((n, HID), jnp.float32)],
    )


@functools.cache
def _build_consume_final(n, nb):
    del nb

    def body(p_ref, b2p, sum_out):
        ht = _agg_to_h_t(p_ref[...], b2p[...])
        sum_out[...] = jnp.sum(ht, axis=1, keepdims=True)

    return pl.pallas_call(
        body,
        out_shape=jax.ShapeDtypeStruct((HID, 1), jnp.float32),
    )


def _heads_body(fsum, ssum, wfp, bfp, wsp, bsp, fw1a, fw1b, fb1, fw2, fb2,
                tmpl, pw1, pb1, pw2, pb2, tw1, tb1, tw2, tb2,
                ew1a, ew1b, eb1, ew2, eb2, exw, exb, etw, etb, ohi, ohj,
                pos_out, typ_out, ex_out, et_out):
    inv_n = 1.0 / N
    fg = _tdot(fsum[...] * inv_n, wfp[...]) + bfp[...]
    sg = _tdot(ssum[...] * inv_n, wsp[...]) + bsp[...]
    pre = jnp.maximum(_mm(fg, fw1a[...]) + _mm(sg, fw1b[...]) + fb1[...], 0.0)
    fused = _mm(pre, fw2[...]) + fb2[...]
    emb = tmpl[...] + fused
    ph = jnp.maximum(_mm(emb, pw1[...]) + pb1[...], 0.0)
    pos_out[...] = _mm(ph, pw2[...]) + pb2[...]
    th = jnp.maximum(_mm(emb, tw1[...]) + tb1[...], 0.0)
    typ_out[...] = _mm(th, tw2[...]) + tb2[...]
    pi = _mm(ohi[...], emb)
    pj = _mm(ohj[...], emb)
    e1 = jnp.maximum(_mm(pi, ew1a[...]) + _mm(pj, ew1b[...]) + eb1[...], 0.0)
    enc = jnp.maximum(_mm(e1, ew2[...]) + eb2[...], 0.0)
    ex_out[...] = _mm(enc, exw[...]) + exb[...]
    et_out[...] = _mm(enc, etw[...]) + etb[...]


@functools.cache
def _build_heads():
    return pl.pallas_call(
        _heads_body,
        out_shape=[jax.ShapeDtypeStruct((6, 2), jnp.float32),
                   jax.ShapeDtypeStruct((6, 2), jnp.float32),
                   jax.ShapeDtypeStruct((15, 1), jnp.float32),
                   jax.ShapeDtypeStruct((15, 16), jnp.float32)],
    )


def _row(v):
    return v.reshape(1, -1)


def _encoder(x, edge_index, enc, n=N, e=E, gc=128, schunk=640, nb=10):
    src = edge_index[0]
    dst = edge_index[1]
    prep = []
    for (cw1, cb1, cw2, cb2) in enc['convs']:
        prep.append((cw1[:HID] - cw1[HID:], _row(cb1), cw1[HID:], cw2,
                     cb2.reshape(-1, 1)))
    w1, b1, w2, b2 = enc['ne']
    a, b = _build_prep0(n, nb)(x, w1, _row(b1), w2, _row(b2),
                               prep[0][0], prep[0][1], prep[0][2])
    gather = _build_gather(n, e, gc)
    edge_mlp = _build_edge_mlp(e, 512)
    scatter = _build_scatter(n, e, schunk)
    for i in range(3):
        z = gather(a, b, dst, src)
        m = edge_mlp(z, prep[i][3])
        p = scatter(m, dst)
        if i < 2:
            a, b = _build_consume(n, nb)(p, prep[i][4], prep[i + 1][0],
                                         prep[i + 1][1], prep[i + 1][2])
        else:
            return _build_consume_final(n, nb)(p, prep[i][4])


def kernel(front_x, front_edge_index, front_edge_attr,
           side_x, side_edge_index, side_edge_attr, params):
    fsum = _encoder(front_x, front_edge_index, params['front'])
    ssum = _encoder(side_x, side_edge_index, params['side'])
    fus_w1, fus_b1, fus_w2, fus_b2 = params['fusion']
    pw1, pb1, pw2, pb2 = params['pos']
    tw1, tb1, tw2, tb2 = params['type']
    ew1, eb1, ew2, eb2 = params['edge']['enc']
    ii, jj = np.triu_indices(6, k=1)
    ohi = jnp.asarray(np.eye(6, dtype=np.float32)[ii])
    ohj = jnp.asarray(np.eye(6, dtype=np.float32)[jj])
    pos, typ, exist, etype = _build_heads()(
        fsum, ssum,
        params['front_pool'][0], _row(params['front_pool'][1]),
        params['side_pool'][0], _row(params['side_pool'][1]),
        fus_w1[:HID], fus_w1[HID:], _row(fus_b1), fus_w2, _row(fus_b2),
        params['templates'][:6],
        pw1, _row(pb1), pw2, _row(pb2),
        tw1, _row(tb1), tw2, _row(tb2),
        ew1[:HID], ew1[HID:], _row(eb1), ew2, _row(eb2),
        params['edge']['exist'][0], _row(params['edge']['exist'][1]),
        params['edge']['type'][0], _row(params['edge']['type'][1]),
        ohi, ohj)
    return pos, typ, exist, etype


# speedup vs baseline: 1.5232x; 1.0026x over previous
import functools

import numpy as np
import jax
import jax.numpy as jnp
from jax import lax
from jax.experimental import pallas as pl
from jax.experimental.pallas import tpu as pltpu
from jax.experimental.pallas import tpu_sc as plsc

N = 10000
E = 320000
HID = 128
NC = 2
NS = 16
NW = NC * NS
F = HID // NS
FMAX = 3.4028235e38
NEG_INF = float("-inf")

_mesh = plsc.VectorSubcoreMesh(
    core_axis_name="c", subcore_axis_name="s", num_cores=NC, num_subcores=NS)


def _mm(a, b):
    return jnp.dot(a, b, preferred_element_type=jnp.float32)


@functools.cache
def _build_gather(n, e, gc):
    ec = e // NW
    nbuf = 4
    sc_sz = nbuf * gc
    nsuper = ec // sc_sz
    rem = ec - nsuper * sc_sz

    @functools.partial(
        pl.kernel,
        out_type=jax.ShapeDtypeStruct((e, HID), jnp.float32),
        mesh=_mesh,
        scratch_types=[
            pltpu.VMEM((ec,), jnp.int32),
            pltpu.VMEM((ec,), jnp.int32),
            pltpu.VMEM((nbuf, gc, HID), jnp.float32),
            pltpu.SemaphoreType.DMA,
            pltpu.SemaphoreType.DMA,
            pltpu.SemaphoreType.DMA,
            pltpu.SemaphoreType.DMA,
        ],
        compiler_params=pltpu.CompilerParams(needs_layout_passes=False),
        name="sc_gather_add",
    )
    def gather(a_hbm, b_hbm, dst_hbm, src_hbm, z_hbm,
               didx, sidx, zbufs, sem_i, sem_a, sem_b, sem_w):
        w = lax.axis_index("c") * NS + lax.axis_index("s")
        base0 = w * ec
        di = pltpu.async_copy(dst_hbm.at[pl.ds(base0, ec)], didx, sem_i)
        si = pltpu.async_copy(src_hbm.at[pl.ds(base0, ec)], sidx, sem_i)
        di.wait()
        si.wait()

        def superchunk(i, carry):
            off = i * sc_sz

            @pl.when(i > 0)
            def _():
                for b in range(nbuf):
                    pltpu.make_async_copy(
                        zbufs.at[b],
                        z_hbm.at[pl.ds(base0 + off - sc_sz + b * gc, gc)],
                        sem_w).wait()

            cps = [pltpu.async_copy(
                a_hbm.at[didx.at[pl.ds(off + b * gc, gc)]],
                zbufs.at[b], sem_a) for b in range(nbuf)]
            for cp in cps:
                cp.wait()
            cps = [pltpu.async_copy(
                b_hbm.at[sidx.at[pl.ds(off + b * gc, gc)]],
                zbufs.at[b], sem_b, add=True) for b in range(nbuf)]
            for cp in cps:
                cp.wait()
            for b in range(nbuf):
                pltpu.async_copy(
                    zbufs.at[b],
                    z_hbm.at[pl.ds(base0 + off + b * gc, gc)], sem_w)
            return carry

        lax.fori_loop(0, nsuper, superchunk, 0)
        for b in range(nbuf):
            pltpu.make_async_copy(
                zbufs.at[b],
                z_hbm.at[pl.ds(base0 + (nsuper - 1) * sc_sz + b * gc, gc)],
                sem_w).wait()
        off = nsuper * sc_sz
        while off < ec:
            sz = min(gc, ec - off)
            zb = zbufs.at[0].at[pl.ds(0, sz)] if sz != gc else zbufs.at[0]
            pltpu.async_copy(a_hbm.at[didx.at[pl.ds(off, sz)]], zb,
                             sem_a).wait()
            pltpu.async_copy(b_hbm.at[sidx.at[pl.ds(off, sz)]], zb,
                             sem_b, add=True).wait()
            pltpu.async_copy(zb, z_hbm.at[pl.ds(base0 + off, sz)],
                             sem_w).wait()
            off += sz

    return gather


@functools.cache
def _build_scatter(n, e, schunk):
    eh = e // 2
    nchunk = eh // schunk
    ngroups = schunk // 16

    @functools.partial(
        pl.kernel,
        out_type=jax.ShapeDtypeStruct((2, HID, n), jnp.float32),
        mesh=_mesh,
        scratch_types=[
            pltpu.VMEM((schunk,), jnp.int32),
            pltpu.VMEM((schunk,), jnp.int32),
            pltpu.VMEM((F, schunk), jnp.float32),
            pltpu.VMEM((F, schunk), jnp.float32),
            pltpu.VMEM((F, n), jnp.float32),
            pltpu.VMEM((n,), jnp.int32),
            pltpu.SemaphoreType.DMA,
            pltpu.SemaphoreType.DMA,
        ],
        compiler_params=pltpu.CompilerParams(needs_layout_passes=False),
        name="sc_scatter_max",
    )
    def scatter(m_hbm, dst_hbm, p_hbm, dbuf0, dbuf1, vbuf0, vbuf1, acc,
                cntbuf, sem0, sem1):
        c = lax.axis_index("c")
        s = lax.axis_index("s")
        iota = lax.iota(jnp.int32, 16)
        neg = jnp.full((16,), NEG_INF, jnp.float32)
        ones = jnp.full((16,), 1, jnp.int32)
        zeros = jnp.full((16,), 0, jnp.int32)

        def initb(i, carry):
            for f in range(F):
                acc[f, pl.ds(i * 16, 16)] = neg
            return carry

        lax.fori_loop(0, n // 16, initb, 0)

        base0 = c * eh
        col = s * F
        fsplat = [jnp.full((16,), f, jnp.int32) for f in range(F)]
        bufs = [(dbuf0, vbuf0, sem0), (dbuf1, vbuf1, sem1)]

        def start_fetch(ci, b):
            db, vb, sem = bufs[b]
            base = base0 + ci * schunk
            pltpu.async_copy(dst_hbm.at[pl.ds(base, schunk)], db, sem)
            pltpu.async_copy(m_hbm.at[pl.ds(col, F), pl.ds(base, schunk)],
                             vb, sem)

        def wait_fetch(b):
            db, vb, sem = bufs[b]
            pltpu.make_async_copy(dst_hbm.at[pl.ds(0, schunk)], db,
                                  sem).wait()
            pltpu.make_async_copy(m_hbm.at[pl.ds(0, F), pl.ds(0, schunk)],
                                  vb, sem).wait()

        def process(b):
            dbuf, vbuf, _ = bufs[b]

            def pairgroup(gp, gcarry):
                datas = []
                lor = None
                for k in range(4):
                    d16 = dbuf[pl.ds(gp * 64 + k * 16, 16)]
                    plsc.store_scatter(cntbuf, [d16], iota)
                    back = plsc.load_gather(cntbuf, [d16])
                    losers = back != iota
                    for f in range(F):
                        vals = vbuf[f, pl.ds(gp * 64 + k * 16, 16)]
                        cur = plsc.load_gather(acc, [fsplat[f], d16])
                        plsc.store_scatter(acc, [fsplat[f], d16],
                                           jnp.maximum(cur, vals))
                    datas.append((d16, losers))
                    lor = losers if k == 0 else lor | losers
                ncol = jnp.max(jnp.where(lor, ones, zeros))

                @pl.when(ncol > 0)
                def _():
                    for k in range(4):
                        d16, losers = datas[k]
                        vals = [vbuf[f, pl.ds(gp * 64 + k * 16, 16)]
                                for f in range(F)]
                        plsc.store_scatter(cntbuf, [d16],
                                           jnp.full((16,), -1, jnp.int32),
                                           mask=losers)
                        mk = plsc.load_gather(cntbuf, [d16])
                        pend0 = mk == -1

                        def cond(carry2):
                            pend, r = carry2
                            return (jnp.max(
                                jnp.where(pend, ones, zeros)) > 0) & (r < 32)

                        def body(carry2):
                            pend, r = carry2
                            pnew = pend & (iota < 0)
                            for f in range(F):
                                cur = plsc.load_gather(
                                    acc, [fsplat[f], d16], mask=pend)
                                nv = jnp.maximum(cur, vals[f])
                                plsc.store_scatter(acc, [fsplat[f], d16],
                                                   nv, mask=pend)
                                chk = plsc.load_gather(
                                    acc, [fsplat[f], d16], mask=pend)
                                pnew = pnew | (pend & (chk < nv))
                            return pnew, r + 1

                        lax.while_loop(cond, body, (pend0, 0))
                return gcarry

            lax.fori_loop(0, ngroups // 4, pairgroup, 0)

        start_fetch(0, 0)

        def pair(j, carry):
            c0 = 2 * j
            wait_fetch(0)
            start_fetch(jnp.minimum(c0 + 1, nchunk - 1), 1)
            process(0)
            wait_fetch(1)
            start_fetch(jnp.minimum(c0 + 2, nchunk - 1), 0)
            process(1)
            return carry

        lax.fori_loop(0, nchunk // 2, pair, 0)
        wait_fetch(0)
        pltpu.sync_copy(acc, p_hbm.at[c, pl.ds(col, F), :])

    return scatter


@functools.cache
def _build_prep0(n, nb):
    bs = n // nb

    def body(x_ref, w1, b1, w2, b2, wa, ba, wb, a_out, b_out):
        x = x_ref[...]
        h = _mm(jnp.maximum(_mm(x, w1[...]) + b1[...], 0.0), w2[...]) + b2[...]
        a_out[...] = _mm(h, wa[...]) + ba[...]
        b_out[...] = _mm(h, wb[...])

    full = pl.BlockSpec((HID, HID), lambda i: (0, 0))
    vec = pl.BlockSpec((1, HID), lambda i: (0, 0))
    return pl.pallas_call(
        body,
        grid=(nb,),
        in_specs=[pl.BlockSpec((bs, HID), lambda i: (i, 0)),
                  full, vec, full, vec, full, vec, full],
        out_specs=[pl.BlockSpec((bs, HID), lambda i: (i, 0)),
                   pl.BlockSpec((bs, HID), lambda i: (i, 0))],
        out_shape=[jax.ShapeDtypeStruct((n, HID), jnp.float32),
                   jax.ShapeDtypeStruct((n, HID), jnp.float32)],
    )


def _tdot(a, b):
    return lax.dot_general(a, b, (((0,), (0,)), ((), ())),
                           preferred_element_type=jnp.float32)


@functools.cache
def _build_edge_mlp(e, bs):
    def body(z_ref, w2, m_ref):
        m_ref[...] = lax.dot_general(
            w2[...], jnp.maximum(z_ref[...], 0.0),
            (((0,), (1,)), ((), ())), preferred_element_type=jnp.float32)

    return pl.pallas_call(
        body,
        grid=(e // bs,),
        in_specs=[pl.BlockSpec((bs, HID), lambda i: (i, 0)),
                  pl.BlockSpec((HID, HID), lambda i: (0, 0))],
        out_specs=pl.BlockSpec((HID, bs), lambda i: (0, i)),
        out_shape=jax.ShapeDtypeStruct((HID, e), jnp.float32),
    )


def _agg_to_h_t(p_blk, b2col):
    aggb = jnp.max(p_blk, axis=0) + b2col
    finite = (aggb >= -FMAX) & (aggb <= FMAX)
    return jnp.maximum(jnp.where(finite, aggb, 0.0), 0.0)


@functools.cache
def _build_consume(n, nb):
    del nb

    def body(p_ref, b2p, wa, ba, wb, a_out, b_out):
        ht = _agg_to_h_t(p_ref[...], b2p[...])
        a_out[...] = _tdot(ht, wa[...]) + ba[...]
        b_out[...] = _tdot(ht, wb[...])

    return pl.pallas_call(
        body,
        out_shape=[jax.ShapeDtypeStruct((n, HID), jnp.float32),
                   jax.ShapeDtypeStruct((n, HID), jnp.float32)],
    )


@functools.cache
def _build_consume_final(n, nb):
    del nb

    def body(p_ref, b2p, sum_out):
        ht = _agg_to_h_t(p_ref[...], b2p[...])
        sum_out[...] = jnp.sum(ht, axis=1, keepdims=True)

    return pl.pallas_call(
        body,
        out_shape=jax.ShapeDtypeStruct((HID, 1), jnp.float32),
    )


def _heads_body(fsum, ssum, wfp, bfp, wsp, bsp, fw1a, fw1b, fb1, fw2, fb2,
                tmpl, pw1, pb1, pw2, pb2, tw1, tb1, tw2, tb2,
                ew1a, ew1b, eb1, ew2, eb2, exw, exb, etw, etb, ohi, ohj,
                pos_out, typ_out, ex_out, et_out):
    inv_n = 1.0 / N
    fg = _tdot(fsum[...] * inv_n, wfp[...]) + bfp[...]
    sg = _tdot(ssum[...] * inv_n, wsp[...]) + bsp[...]
    pre = jnp.maximum(_mm(fg, fw1a[...]) + _mm(sg, fw1b[...]) + fb1[...], 0.0)
    fused = _mm(pre, fw2[...]) + fb2[...]
    emb = tmpl[...] + fused
    ph = jnp.maximum(_mm(emb, pw1[...]) + pb1[...], 0.0)
    pos_out[...] = _mm(ph, pw2[...]) + pb2[...]
    th = jnp.maximum(_mm(emb, tw1[...]) + tb1[...], 0.0)
    typ_out[...] = _mm(th, tw2[...]) + tb2[...]
    pi = _mm(ohi[...], emb)
    pj = _mm(ohj[...], emb)
    e1 = jnp.maximum(_mm(pi, ew1a[...]) + _mm(pj, ew1b[...]) + eb1[...], 0.0)
    enc = jnp.maximum(_mm(e1, ew2[...]) + eb2[...], 0.0)
    ex_out[...] = _mm(enc, exw[...]) + exb[...]
    et_out[...] = _mm(enc, etw[...]) + etb[...]


@functools.cache
def _build_heads():
    return pl.pallas_call(
        _heads_body,
        out_shape=[jax.ShapeDtypeStruct((6, 2), jnp.float32),
                   jax.ShapeDtypeStruct((6, 2), jnp.float32),
                   jax.ShapeDtypeStruct((15, 1), jnp.float32),
                   jax.ShapeDtypeStruct((15, 16), jnp.float32)],
    )


def _row(v):
    return v.reshape(1, -1)


def _encoder(x, edge_index, enc, n=N, e=E, gc=128, schunk=640, nb=10):
    src = edge_index[0]
    dst = edge_index[1]
    prep = []
    for (cw1, cb1, cw2, cb2) in enc['convs']:
        prep.append((cw1[:HID] - cw1[HID:], _row(cb1), cw1[HID:], cw2,
                     cb2.reshape(-1, 1)))
    w1, b1, w2, b2 = enc['ne']
    a, b = _build_prep0(n, nb)(x, w1, _row(b1), w2, _row(b2),
                               prep[0][0], prep[0][1], prep[0][2])
    gather = _build_gather(n, e, gc)
    edge_mlp = _build_edge_mlp(e, 512)
    scatter = _build_scatter(n, e, schunk)
    for i in range(3):
        z = gather(a, b, dst, src)
        m = edge_mlp(z, prep[i][3])
        p = scatter(m, dst)
        if i < 2:
            a, b = _build_consume(n, nb)(p, prep[i][4], prep[i + 1][0],
                                         prep[i + 1][1], prep[i + 1][2])
        else:
            return _build_consume_final(n, nb)(p, prep[i][4])


def kernel(front_x, front_edge_index, front_edge_attr,
           side_x, side_edge_index, side_edge_attr, params):
    fsum = _encoder(front_x, front_edge_index, params['front'])
    ssum = _encoder(side_x, side_edge_index, params['side'])
    fus_w1, fus_b1, fus_w2, fus_b2 = params['fusion']
    pw1, pb1, pw2, pb2 = params['pos']
    tw1, tb1, tw2, tb2 = params['type']
    ew1, eb1, ew2, eb2 = params['edge']['enc']
    ii, jj = np.triu_indices(6, k=1)
    ohi = jnp.asarray(np.eye(6, dtype=np.float32)[ii])
    ohj = jnp.asarray(np.eye(6, dtype=np.float32)[jj])
    pos, typ, exist, etype = _build_heads()(
        fsum, ssum,
        params['front_pool'][0], _row(params['front_pool'][1]),
        params['side_pool'][0], _row(params['side_pool'][1]),
        fus_w1[:HID], fus_w1[HID:], _row(fus_b1), fus_w2, _row(fus_b2),
        params['templates'][:6],
        pw1, _row(pb1), pw2, _row(pb2),
        tw1, _row(tb1), tw2, _row(tb2),
        ew1[:HID], ew1[HID:], _row(eb1), ew2, _row(eb2),
        params['edge']['exist'][0], _row(params['edge']['exist'][1]),
        params['edge']['type'][0], _row(params['edge']['type'][1]),
        ohi, ohj)
    return pos, typ, exist, etype
